# baseline (speedup 1.0000x reference)
import jax
import jax.numpy as jnp
from jax import lax
from jax.experimental import pallas as pl
from jax.experimental.pallas import tpu as pltpu
from jax.experimental.pallas import tpu_sc as plsc

N_VOCAB = 100000
N_HEADS = 8
N_FEATURES = 64
BATCH = 4096
HIST = 50

BTILE = 128
FT = N_FEATURES // 8
TILE_ELEMS = 8 * BTILE
GSLOT = N_HEADS
SSLOT = 4
LOOKAHEAD = 4
L = 16


def _make_kernel():
    info = plsc.get_sparse_core_info()
    nc, ns = info.num_cores, info.num_subcores
    nw = nc * ns
    assert BATCH // BTILE == nw

    mesh = plsc.VectorSubcoreMesh(core_axis_name="c", subcore_axis_name="s")

    def body(seq_hbm, tab_hbm, out_hbm, seq_v, base_v, idx_v, rows_v,
             tbuf_v, gsem, ssem):
        wid = lax.axis_index("s") * nc + lax.axis_index("c")
        seq_per_tile = BTILE * HIST
        pltpu.sync_copy(seq_hbm.at[pl.ds(wid * seq_per_tile, seq_per_tile)],
                        seq_v)
        iota = lax.iota(jnp.int32, L)
        scat = [iota * BTILE + (2048 * k) for k in range(4)]

        def drain_gather(gslot):
            pltpu.make_async_copy(
                tab_hbm.at[pl.ds(0, BTILE)], rows_v.at[gslot],
                gsem.at[gslot]).wait()

        def drain_stores(sslot):
            for ft in range(FT):
                pltpu.make_async_copy(
                    out_hbm.at[0, 0, ft, wid],
                    tbuf_v.at[sslot, pl.ds(ft * TILE_ELEMS, TILE_ELEMS)],
                    ssem.at[sslot]).wait()

        def fire_gather(gslot, h):
            off = jnp.int32(h * N_VOCAB)
            for j in range(BTILE // L):
                idx_v[gslot, pl.ds(j * L, L)] = base_v[pl.ds(j * L, L)] + off
            pltpu.async_copy(tab_hbm.at[idx_v.at[gslot]], rows_v.at[gslot],
                             gsem.at[gslot])

        def transpose_and_store(gslot, sslot, h_prev, t_prev):
            drain_gather(gslot)

            @plsc.parallel_loop(0, BTILE, step=8, unroll=2)
            def _(c):
                for bi in range(8):
                    b = c + bi
                    for k in range(4):
                        v = rows_v[gslot, b, pl.ds(k * L, L)]
                        plsc.store_scatter(
                            tbuf_v.at[sslot], [scat[k] + b], v)

            for ft in range(FT):
                pltpu.async_copy(
                    tbuf_v.at[sslot, pl.ds(ft * TILE_ELEMS, TILE_ELEMS)],
                    out_hbm.at[h_prev, t_prev, ft, wid],
                    ssem.at[sslot])

        def retire(s_h, t):
            h_prev = (s_h - LOOKAHEAD) % N_HEADS
            t_prev = jnp.where(s_h >= LOOKAHEAD, t, t - 1)
            sslot = (s_h - LOOKAHEAD) % SSLOT

            @pl.when(t > 0)
            def _():
                drain_stores(sslot)
            transpose_and_store(h_prev, sslot, h_prev, t_prev)

        def t_body(t, carry):
            for j in range(BTILE // L):
                base_v[pl.ds(j * L, L)] = plsc.load_gather(
                    seq_v, [iota * HIST + (j * L * HIST + t)])
            for h in range(N_HEADS):
                fire_gather(h, h)
                if h >= LOOKAHEAD:
                    retire(h, t)
                else:
                    @pl.when(t > 0)
                    def _():
                        retire(h, t)
            return carry

        lax.fori_loop(0, HIST, t_body, 0)
        for s_h in range(N_HEADS, N_HEADS + LOOKAHEAD):
            h_prev = (s_h - LOOKAHEAD) % N_HEADS
            sslot = (s_h - LOOKAHEAD) % SSLOT
            drain_stores(sslot)
            transpose_and_store(h_prev, sslot, h_prev, HIST - 1)
        for sslot in range(SSLOT):
            drain_stores(sslot)

    return pl.kernel(
        body,
        out_type=jax.ShapeDtypeStruct(
            (N_HEADS, HIST, FT, BATCH // BTILE, TILE_ELEMS), jnp.float32),
        mesh=mesh,
        scratch_types=[
            pltpu.VMEM((BTILE * HIST,), jnp.int32),
            pltpu.VMEM((BTILE,), jnp.int32),
            pltpu.VMEM((GSLOT, BTILE), jnp.int32),
            pltpu.VMEM((GSLOT, BTILE, N_FEATURES), jnp.float32),
            pltpu.VMEM((SSLOT, FT * TILE_ELEMS), jnp.float32),
            pltpu.SemaphoreType.DMA((GSLOT,)),
            pltpu.SemaphoreType.DMA((SSLOT,)),
        ],
        compiler_params=pltpu.CompilerParams(
            use_tc_tiling_on_sc=False, needs_layout_passes=False),
    )


def kernel(seq, tables):
    seq_flat = seq.reshape(-1).astype(jnp.int32)
    tab_flat = tables.reshape(N_HEADS * N_VOCAB, N_FEATURES)
    out = _make_kernel()(seq_flat, tab_flat)
    return (out.reshape(N_HEADS, HIST, FT, BATCH // BTILE, 8, BTILE)
            .transpose(3, 5, 0, 1, 2, 4)
            .reshape(BATCH, N_HEADS, HIST, N_FEATURES))


# speedup vs baseline: 1.2885x; 1.2885x over previous
import jax
import jax.numpy as jnp
from jax import lax
from jax.experimental import pallas as pl
from jax.experimental.pallas import tpu as pltpu
from jax.experimental.pallas import tpu_sc as plsc

N_VOCAB = 100000
N_HEADS = 8
N_FEATURES = 64
BATCH = 4096
HIST = 50

BTILE = 128
FT = N_FEATURES // 8
TILE_ELEMS = 8 * BTILE
GSLOT = N_HEADS
SSLOT = 4
LOOKAHEAD = 4
UNITS = HIST * N_HEADS
L = 16


def _make_kernel():
    info = plsc.get_sparse_core_info()
    nc, ns = info.num_cores, info.num_subcores
    nw = nc * ns
    assert BATCH // BTILE == nw

    mesh = plsc.VectorSubcoreMesh(core_axis_name="c", subcore_axis_name="s")

    def body(seq_hbm, tab_hbm, out_hbm, seq_v, base_v, idx_v, rows_v,
             tbuf_v, gsem, ssem):
        wid = lax.axis_index("s") * nc + lax.axis_index("c")
        seq_per_tile = BTILE * HIST
        pltpu.sync_copy(seq_hbm.at[pl.ds(wid * seq_per_tile, seq_per_tile)],
                        seq_v)
        iota = lax.iota(jnp.int32, L)
        diag_f = [(iota + s) & 15 for s in range(L)]
        diag_s = [(((iota + s) & 15) * BTILE) + iota for s in range(L)]

        def drain_stores(sslot):
            for ft in range(FT):
                pltpu.make_async_copy(
                    out_hbm.at[0, 0, ft, wid],
                    tbuf_v.at[sslot, pl.ds(ft * TILE_ELEMS, TILE_ELEMS)],
                    ssem.at[sslot]).wait()

        def u_body(u, carry):
            h = lax.rem(u, N_HEADS)
            t = lax.div(u, N_HEADS)

            @pl.when(jnp.logical_and(u < UNITS, h == 0))
            def _():
                for j in range(BTILE // L):
                    base_v[pl.ds(j * L, L)] = plsc.load_gather(
                        seq_v, [iota * HIST + (j * L * HIST + t)])

            @pl.when(u < UNITS)
            def _():
                off = h * N_VOCAB
                for j in range(BTILE // L):
                    idx_v[h, pl.ds(j * L, L)] = (
                        base_v[pl.ds(j * L, L)] + off)
                pltpu.async_copy(tab_hbm.at[idx_v.at[h]], rows_v.at[h],
                                 gsem.at[h])

            @pl.when(u >= LOOKAHEAD)
            def _():
                up = u - LOOKAHEAD
                hp = lax.rem(up, N_HEADS)
                tp = lax.div(up, N_HEADS)
                ss = lax.rem(up, SSLOT)

                @pl.when(u >= LOOKAHEAD + SSLOT)
                def _():
                    drain_stores(ss)
                pltpu.make_async_copy(
                    tab_hbm.at[pl.ds(0, BTILE)], rows_v.at[hp],
                    gsem.at[hp]).wait()

                @plsc.parallel_loop(0, BTILE // L, step=1, unroll=2)
                def _(j):
                    bbase = j * L
                    idx_b = iota + bbase
                    for k in range(N_FEATURES // L):
                        for s in range(L):
                            v = plsc.load_gather(
                                rows_v.at[hp], [idx_b, diag_f[s] + (k * L)])
                            plsc.store_scatter(
                                tbuf_v.at[ss],
                                [diag_s[s] + (k * L * BTILE + bbase)], v)

                for ft in range(FT):
                    pltpu.async_copy(
                        tbuf_v.at[ss, pl.ds(ft * TILE_ELEMS, TILE_ELEMS)],
                        out_hbm.at[hp, tp, ft, wid],
                        ssem.at[ss])

            return carry

        lax.fori_loop(0, UNITS + LOOKAHEAD, u_body, 0)
        for sslot in range(SSLOT):
            drain_stores(sslot)

    return pl.kernel(
        body,
        out_type=jax.ShapeDtypeStruct(
            (N_HEADS, HIST, FT, BATCH // BTILE, TILE_ELEMS), jnp.float32),
        mesh=mesh,
        scratch_types=[
            pltpu.VMEM((BTILE * HIST,), jnp.int32),
            pltpu.VMEM((BTILE,), jnp.int32),
            pltpu.VMEM((GSLOT, BTILE), jnp.int32),
            pltpu.VMEM((GSLOT, BTILE, N_FEATURES), jnp.float32),
            pltpu.VMEM((SSLOT, FT * TILE_ELEMS), jnp.float32),
            pltpu.SemaphoreType.DMA((GSLOT,)),
            pltpu.SemaphoreType.DMA((SSLOT,)),
        ],
        compiler_params=pltpu.CompilerParams(
            use_tc_tiling_on_sc=False, needs_layout_passes=False),
    )


def kernel(seq, tables):
    seq_flat = seq.reshape(-1).astype(jnp.int32)
    tab_flat = tables.reshape(N_HEADS * N_VOCAB, N_FEATURES)
    out = _make_kernel()(seq_flat, tab_flat)
    return (out.reshape(N_HEADS, HIST, FT, BATCH // BTILE, 8, BTILE)
            .transpose(3, 5, 0, 1, 2, 4)
            .reshape(BATCH, N_HEADS, HIST, N_FEATURES))


# speedup vs baseline: 2.5392x; 1.9706x over previous
import jax
import jax.numpy as jnp
from jax import lax
from jax.experimental import pallas as pl
from jax.experimental.pallas import tpu as pltpu
from jax.experimental.pallas import tpu_sc as plsc

N_VOCAB = 100000
N_HEADS = 8
N_FEATURES = 64
BATCH = 4096
HIST = 50

BTILE = 128
FT = N_FEATURES // 8
TILE_ELEMS = 8 * BTILE
GSLOT = N_HEADS
SSLOT = 4
LOOKAHEAD = 4
UNITS = HIST * N_HEADS
L = 16


def _make_kernel():
    info = plsc.get_sparse_core_info()
    nc, ns = info.num_cores, info.num_subcores
    nw = nc * ns
    assert BATCH // BTILE == nw

    mesh = plsc.VectorSubcoreMesh(core_axis_name="c", subcore_axis_name="s")

    def body(seq_hbm, tab_hbm, out_hbm, seq_v, base_v, idx_v, rows_v,
             tbuf_v, gsem, ssem):
        wid = lax.axis_index("s") * nc + lax.axis_index("c")
        seq_per_tile = BTILE * HIST
        pltpu.sync_copy(seq_hbm.at[pl.ds(wid * seq_per_tile, seq_per_tile)],
                        seq_v)
        iota = lax.iota(jnp.int32, L)
        diag_f = [(iota + s) & 15 for s in range(L)]
        diag_s = [(((iota + s) & 15) * BTILE) + iota for s in range(L)]

        def drain_stores(sslot):
            for ft in range(FT):
                pltpu.make_async_copy(
                    out_hbm.at[0, 0, ft, wid],
                    tbuf_v.at[sslot, pl.ds(ft * TILE_ELEMS, TILE_ELEMS)],
                    ssem.at[sslot]).wait()

        def u_body(u, carry):
            h = lax.rem(u, N_HEADS)
            t = lax.div(u, N_HEADS)

            @pl.when(jnp.logical_and(u < UNITS, h == 0))
            def _():
                for j in range(BTILE // L):
                    base_v[pl.ds(j * L, L)] = plsc.load_gather(
                        seq_v, [iota * HIST + (j * L * HIST + t)])

            @pl.when(u < UNITS)
            def _():
                off = h * N_VOCAB
                for j in range(BTILE // L):
                    idx_v[h, pl.ds(j * L, L)] = (
                        base_v[pl.ds(j * L, L)] + off)
                pltpu.async_copy(tab_hbm.at[idx_v.at[h]], rows_v.at[h],
                                 gsem.at[h])

            @pl.when(u >= LOOKAHEAD)
            def _():
                up = u - LOOKAHEAD
                hp = lax.rem(up, N_HEADS)
                tp = lax.div(up, N_HEADS)
                ss = lax.rem(up, SSLOT)

                @pl.when(u >= LOOKAHEAD + SSLOT)
                def _():
                    drain_stores(ss)
                pltpu.make_async_copy(
                    tab_hbm.at[pl.ds(0, BTILE)], rows_v.at[hp],
                    gsem.at[hp]).wait()

                @plsc.parallel_loop(0, BTILE // L, step=1, unroll=4)
                def _(j):
                    bbase = j * L
                    idx_b = iota + bbase
                    for k in range(N_FEATURES // L):
                        for s in range(L):
                            v = plsc.load_gather(
                                rows_v.at[hp], [idx_b, diag_f[s] + (k * L)])
                            plsc.store_scatter(
                                tbuf_v.at[ss],
                                [diag_s[s] + (k * L * BTILE + bbase)], v)

                for ft in range(FT):
                    pltpu.async_copy(
                        tbuf_v.at[ss, pl.ds(ft * TILE_ELEMS, TILE_ELEMS)],
                        out_hbm.at[hp, tp, ft, wid],
                        ssem.at[ss])

            return carry

        lax.fori_loop(0, UNITS + LOOKAHEAD, u_body, 0)
        for sslot in range(SSLOT):
            drain_stores(sslot)

    return pl.kernel(
        body,
        out_type=jax.ShapeDtypeStruct(
            (N_HEADS, HIST, FT, BATCH // BTILE, TILE_ELEMS), jnp.float32),
        mesh=mesh,
        scratch_types=[
            pltpu.VMEM((BTILE * HIST,), jnp.int32),
            pltpu.VMEM((BTILE,), jnp.int32),
            pltpu.VMEM((GSLOT, BTILE), jnp.int32),
            pltpu.VMEM((GSLOT, BTILE, N_FEATURES), jnp.float32),
            pltpu.VMEM((SSLOT, FT * TILE_ELEMS), jnp.float32),
            pltpu.SemaphoreType.DMA((GSLOT,)),
            pltpu.SemaphoreType.DMA((SSLOT,)),
        ],
        compiler_params=pltpu.CompilerParams(
            use_tc_tiling_on_sc=False, needs_layout_passes=False),
    )


def kernel(seq, tables):
    seq_flat = seq.reshape(-1).astype(jnp.int32)
    tab_flat = tables.reshape(N_HEADS * N_VOCAB, N_FEATURES)
    out = _make_kernel()(seq_flat, tab_flat)
    return (out.reshape(N_HEADS, HIST, FT, BATCH // BTILE, 8, BTILE)
            .transpose(3, 5, 0, 1, 2, 4)
            .reshape(BATCH, N_HEADS, HIST, N_FEATURES))
